# Initial kernel scaffold; baseline (speedup 1.0000x reference)
import jax
import jax.numpy as jnp
from jax.experimental import pallas as pl


def kernel(x, edge_index, edge_attr, u, batch, W, b):
    raise NotImplementedError("write your pallas kernel here")


# speedup vs baseline: 3.7162x; 3.7162x over previous
import functools

import jax
import jax.numpy as jnp
from jax import lax
from jax.experimental import pallas as pl
from jax.experimental.pallas import tpu as pltpu
from jax.experimental.pallas import tpu_sc as plsc

N = 10000
D = 128
B = 128
NC = 2
NS = 16
NW = NC * NS
L = 16
CB = D // L
NACC = CB + 1
GROUPS = N // L
GBASE = GROUPS // NW
GEXTRA = GROUPS - GBASE * NW
RBASE = GBASE * L
RMAX = (GBASE + 1) * L
CNTB = B
DUMP = 2 * B
SROWS = 264


def _iota():
    return lax.iota(jnp.int32, L)


_mesh = plsc.VectorSubcoreMesh(core_axis_name="c", subcore_axis_name="s")


@functools.partial(
    pl.kernel,
    out_type=jax.ShapeDtypeStruct((NC, 2 * B, D), jnp.float32),
    mesh=_mesh,
    scratch_types=[
        pltpu.VMEM((RMAX, D), jnp.float32),
        pltpu.VMEM((RMAX,), jnp.int32),
        pltpu.VMEM((L, D), jnp.float32),
        pltpu.VMEM((L,), jnp.int32),
        pltpu.VMEM((NACC * L,), jnp.float32),
        pltpu.VMEM((L,), jnp.int32),
        pltpu.VMEM((L,), jnp.int32),
        pltpu.VMEM((8, D), jnp.float32),
        pltpu.VMEM_SHARED((SROWS, D), jnp.float32),
    ],
)
def _seg_sums(x_hbm, batch_hbm, out_hbm,
              xbuf, bbuf, flushbuf, idbuf, accbuf, curbuf, jbuf, zbuf, sacc):
    cid = lax.axis_index("c")
    sid = lax.axis_index("s")
    w = cid * NS + sid

    zero = jnp.zeros((L,), jnp.float32)
    dump_idv = jnp.full((L,), DUMP, jnp.int32)

    for r in range(8):
        for cc in range(CB):
            zbuf[r, pl.ds(cc * L, L)] = zero
    for cc in range(NACC):
        accbuf[pl.ds(cc * L, L)] = zero
    curbuf[...] = jnp.full((L,), -1, jnp.int32)
    jbuf[...] = jnp.zeros((L,), jnp.int32)
    idbuf[...] = dump_idv

    pltpu.sync_copy(zbuf, sacc.at[pl.ds(sid * 16, 8)])
    pltpu.sync_copy(zbuf, sacc.at[pl.ds(sid * 16 + 8, 8)])

    @pl.when(sid == 0)
    def _():
        pltpu.sync_copy(zbuf, sacc.at[pl.ds(256, 8)])

    base = (GBASE * w + jnp.minimum(w, GEXTRA)) * L
    pltpu.sync_copy(x_hbm.at[pl.ds(base, RBASE)], xbuf.at[pl.ds(0, RBASE)])
    pltpu.sync_copy(batch_hbm.at[pl.ds(base, RBASE)], bbuf.at[pl.ds(0, RBASE)])

    @pl.when(w < GEXTRA)
    def _():
        pltpu.sync_copy(x_hbm.at[pl.ds(base + RBASE, L)], xbuf.at[pl.ds(RBASE, L)])
        pltpu.sync_copy(batch_hbm.at[pl.ds(base + RBASE, L)],
                        bbuf.at[pl.ds(RBASE, L)])

    plsc.subcore_barrier()

    def do_flush(seg):
        row_s = jbuf[pl.ds(0, L)][0] % L
        cntv = accbuf[pl.ds(CB * L, L)]
        for cc in range(CB):
            flushbuf[row_s, pl.ds(cc * L, L)] = accbuf[pl.ds(cc * L, L)]
            flushbuf[row_s + 1, pl.ds(cc * L, L)] = cntv
        idv = idbuf[...]
        idv = jnp.where(_iota() == row_s, jnp.full((L,), seg, jnp.int32), idv)
        idv = jnp.where(_iota() == row_s + 1,
                        jnp.full((L,), seg + CNTB, jnp.int32), idv)
        idbuf[...] = idv

        @pl.when(row_s == L - 2)
        def _():
            pltpu.sync_copy(flushbuf, sacc.at[idbuf], add=True)
            idbuf[...] = dump_idv

        jbuf[...] = jbuf[...] + 2
        for cc in range(NACC):
            accbuf[pl.ds(cc * L, L)] = zero

    def step(g, carry):
        rbase = g * L
        bv = bbuf[pl.ds(rbase, L)]
        b0 = bv[0]
        b15 = bv[L - 1]
        cur0 = curbuf[pl.ds(0, L)][0]

        @pl.when(cur0 != b0)
        def _():
            @pl.when(cur0 >= 0)
            def _():
                do_flush(cur0)

        @pl.when(b0 == b15)
        def _():
            accs = [accbuf[pl.ds(cc * L, L)] for cc in range(NACC)]
            for r in range(L):
                for cc in range(CB):
                    accs[cc] = accs[cc] + xbuf[rbase + r, pl.ds(cc * L, L)]
            accs[CB] = accs[CB] + jnp.float32(L)
            for cc in range(NACC):
                accbuf[pl.ds(cc * L, L)] = accs[cc]

        @pl.when(b0 != b15)
        def _():
            for r in range(L):
                if r > 0:
                    @pl.when(bv[r] != bv[r - 1])
                    def _(r=r):
                        do_flush(bv[r - 1])
                for cc in range(CB):
                    accbuf[pl.ds(cc * L, L)] = (
                        accbuf[pl.ds(cc * L, L)] + xbuf[rbase + r, pl.ds(cc * L, L)])
                accbuf[pl.ds(CB * L, L)] = accbuf[pl.ds(CB * L, L)] + jnp.float32(1)

        curbuf[...] = jnp.full((L,), b15, jnp.int32)
        return carry

    ngroups = GBASE + jnp.where(w < GEXTRA, 1, 0)
    lax.fori_loop(0, ngroups, step, jnp.int32(0))

    cur0 = curbuf[pl.ds(0, L)][0]
    do_flush(jnp.where(cur0 < 0, DUMP - CNTB, cur0))
    pltpu.sync_copy(flushbuf, sacc.at[idbuf], add=True)

    plsc.subcore_barrier()

    pltpu.sync_copy(sacc.at[pl.ds(sid * 16, 16)],
                    out_hbm.at[cid, pl.ds(sid * 16, 16)])


def _head_body(p_ref, u_ref, w_ref, b_ref, o_ref):
    s = p_ref[0] + p_ref[1]
    counts = s[CNTB:, 0:1]
    mean = s[:B] / jnp.maximum(counts, 1.0)
    w = w_ref[...]
    h = lax.dot_general(u_ref[...], w[:, :D], (((1,), (1,)), ((), ())),
                        preferred_element_type=jnp.float32)
    h = h + lax.dot_general(mean, w[:, D:], (((1,), (1,)), ((), ())),
                            preferred_element_type=jnp.float32)
    h = h + b_ref[...]
    o_ref[...] = jnp.maximum(h, 0.0)


_head = pl.pallas_call(
    _head_body,
    out_shape=jax.ShapeDtypeStruct((B, D), jnp.float32),
)


def kernel(x, edge_index, edge_attr, u, batch, W, b):
    del edge_index, edge_attr
    parts = _seg_sums(x, batch.astype(jnp.int32))
    return _head(parts, u, W, b.reshape(1, D))


# speedup vs baseline: 5.0026x; 1.3462x over previous
import functools

import jax
import jax.numpy as jnp
from jax import lax
from jax.experimental import pallas as pl
from jax.experimental.pallas import tpu as pltpu
from jax.experimental.pallas import tpu_sc as plsc

N = 10000
D = 128
B = 128
NC = 2
NS = 16
NW = NC * NS
L = 16
CB = D // L
GROUPS = N // L
GBASE = GROUPS // NW
GEXTRA = GROUPS - GBASE * NW
RBASE = GBASE * L
RMAX = (GBASE + 1) * L
CNTB = B
DUMP = 2 * B
SROWS = 264
CHUNK = 4


def _iota():
    return lax.iota(jnp.int32, L)


_mesh = plsc.VectorSubcoreMesh(core_axis_name="c", subcore_axis_name="s")


@functools.partial(
    pl.kernel,
    compiler_params=pltpu.CompilerParams(needs_layout_passes=False),
    out_type=jax.ShapeDtypeStruct((NC, 2 * B, D), jnp.float32),
    mesh=_mesh,
    scratch_types=[
        pltpu.VMEM((RMAX, D), jnp.float32),
        pltpu.VMEM((RMAX,), jnp.int32),
        pltpu.VMEM((L, D), jnp.float32),
        pltpu.VMEM((L,), jnp.int32),
        pltpu.VMEM((8, D), jnp.float32),
        pltpu.SMEM((4,), jnp.int32),
        pltpu.SemaphoreType.DMA,
        pltpu.SemaphoreType.DMA,
        pltpu.VMEM_SHARED((SROWS, D), jnp.float32),
    ],
)
def _seg_sums(x_hbm, batch_hbm, out_hbm,
              xbuf, bbuf, flushbuf, idbuf, zbuf, sm, semx, semb, sacc):
    cid = lax.axis_index("c")
    sid = lax.axis_index("s")
    w = cid * NS + sid

    zero = jnp.zeros((L,), jnp.float32)
    dump_idv = jnp.full((L,), DUMP, jnp.int32)

    base = (GBASE * w + jnp.minimum(w, GEXTRA)) * L
    nfull = RBASE // (CHUNK * L)
    for k in range(nfull):
        pltpu.async_copy(x_hbm.at[pl.ds(base + k * CHUNK * L, CHUNK * L)],
                         xbuf.at[pl.ds(k * CHUNK * L, CHUNK * L)], semx)
    TAIL = RBASE - nfull * CHUNK * L
    pltpu.async_copy(x_hbm.at[pl.ds(base + nfull * CHUNK * L, TAIL)],
                     xbuf.at[pl.ds(nfull * CHUNK * L, TAIL)], semx)

    @pl.when(w < GEXTRA)
    def _():
        pltpu.async_copy(x_hbm.at[pl.ds(base + RBASE, L)],
                         xbuf.at[pl.ds(RBASE, L)], semx)
        pltpu.async_copy(batch_hbm.at[pl.ds(base + RBASE, L)],
                         bbuf.at[pl.ds(RBASE, L)], semb)

    bcp = pltpu.async_copy(batch_hbm.at[pl.ds(base, RBASE)],
                           bbuf.at[pl.ds(0, RBASE)], semb)

    for r in range(8):
        for cc in range(CB):
            zbuf[r, pl.ds(cc * L, L)] = zero
    sm[0] = jnp.int32(-1)
    sm[1] = jnp.int32(0)
    idbuf[...] = dump_idv

    pltpu.sync_copy(zbuf, sacc.at[pl.ds(sid * 16, 8)])
    pltpu.sync_copy(zbuf, sacc.at[pl.ds(sid * 16 + 8, 8)])

    @pl.when(sid == 0)
    def _():
        pltpu.sync_copy(zbuf, sacc.at[pl.ds(256, 8)])

    plsc.subcore_barrier()
    bcp.wait()

    def do_flush(seg, accs):
        row_s = sm[1] % L
        rv0 = jnp.full((L,), row_s, jnp.int32)
        rv1 = rv0 + 1
        for cc in range(CB):
            plsc.store_scatter(flushbuf, [rv0, cc * L + _iota()], accs[cc])
            plsc.store_scatter(flushbuf, [rv1, cc * L + _iota()], accs[CB])
        idv = idbuf[...]
        idv = jnp.where(_iota() == row_s, jnp.full((L,), seg, jnp.int32), idv)
        idv = jnp.where(_iota() == row_s + 1,
                        jnp.full((L,), seg + CNTB, jnp.int32), idv)
        idbuf[...] = idv

        @pl.when(row_s == L - 2)
        def _():
            pltpu.sync_copy(flushbuf, sacc.at[idbuf], add=True)
            idbuf[...] = dump_idv

        sm[1] = sm[1] + 2

    def step(g, accs):
        accs = list(accs)
        rbase = g * L

        @pl.when((g < nfull * CHUNK) & (g % CHUNK == 0))
        def _():
            pltpu.make_async_copy(x_hbm.at[pl.ds(0, CHUNK * L)],
                                  xbuf.at[pl.ds(0, CHUNK * L)], semx).wait()

        @pl.when(g == nfull * CHUNK)
        def _():
            pltpu.make_async_copy(x_hbm.at[pl.ds(0, TAIL)],
                                  xbuf.at[pl.ds(0, TAIL)], semx).wait()

        @pl.when(g == GBASE)
        def _():
            pltpu.make_async_copy(x_hbm.at[pl.ds(0, L)],
                                  xbuf.at[pl.ds(0, L)], semx).wait()
            pltpu.make_async_copy(batch_hbm.at[pl.ds(0, L)],
                                  bbuf.at[pl.ds(0, L)], semb).wait()

        bv = bbuf[pl.ds(rbase, L)]
        b = [bv[r] for r in range(L)]
        cur0 = sm[0]

        @pl.when(cur0 != b[0])
        def _():
            @pl.when(cur0 >= 0)
            def _():
                do_flush(cur0, accs)

        snapshots = []
        run = accs
        boundary0 = cur0 != b[0]
        for r in range(L):
            rows_r = [xbuf[rbase + r, pl.ds(cc * L, L)] for cc in range(CB)]
            boundary = boundary0 if r == 0 else (b[r] != b[r - 1])
            if r > 0:
                snapshots.append((boundary, b[r - 1], run))
            bvp = jnp.full((L,), boundary)
            nxt = [jnp.where(bvp, rw, a + rw) for a, rw in zip(run, rows_r)]
            nxt.append(jnp.where(bvp, jnp.float32(1), run[CB] + 1))
            run = nxt

        @pl.when(b[0] != b[L - 1])
        def _():
            for boundary, seg, snap in snapshots:
                @pl.when(boundary)
                def _(seg=seg, snap=snap):
                    do_flush(seg, snap)

        sm[0] = b[L - 1]
        return tuple(run)

    ngroups = GBASE + jnp.where(w < GEXTRA, 1, 0)
    init = tuple(jnp.zeros((L,), jnp.float32) for _ in range(CB + 1))
    accs = lax.fori_loop(0, ngroups, step, init)

    cur0 = sm[0]
    do_flush(jnp.where(cur0 < 0, DUMP - CNTB, cur0), list(accs))
    pltpu.sync_copy(flushbuf, sacc.at[idbuf], add=True)

    plsc.subcore_barrier()

    pltpu.sync_copy(sacc.at[pl.ds(sid * 16, 16)],
                    out_hbm.at[cid, pl.ds(sid * 16, 16)])


def _head_body(p_ref, u_ref, w_ref, b_ref, o_ref):
    s = p_ref[0] + p_ref[1]
    counts = s[CNTB:, 0:1]
    mean = s[:B] / jnp.maximum(counts, 1.0)
    w = w_ref[...]
    h = lax.dot_general(u_ref[...], w[:, :D], (((1,), (1,)), ((), ())),
                        preferred_element_type=jnp.float32)
    h = h + lax.dot_general(mean, w[:, D:], (((1,), (1,)), ((), ())),
                            preferred_element_type=jnp.float32)
    h = h + b_ref[...]
    o_ref[...] = jnp.maximum(h, 0.0)


_head = pl.pallas_call(
    _head_body,
    out_shape=jax.ShapeDtypeStruct((B, D), jnp.float32),
)


def kernel(x, edge_index, edge_attr, u, batch, W, b):
    del edge_index, edge_attr
    parts = _seg_sums(x, batch.astype(jnp.int32))
    return _head(parts, u, W, b.reshape(1, D))
